# baseline (speedup 1.0000x reference)
import functools

import jax
import jax.numpy as jnp
from jax import lax
from jax.experimental import pallas as pl
from jax.experimental.pallas import tpu as pltpu
from jax.experimental.pallas import tpu_sc as plsc


def kernel(batch_ref, table):
    B = batch_ref.shape[0]
    Q, H = table.shape
    info = plsc.get_sparse_core_info()
    NC, NS = info.num_cores, info.num_subcores
    NW = NC * NS
    b_per_w = B // NW

    mesh = plsc.VectorSubcoreMesh(core_axis_name="c", subcore_axis_name="s")

    @functools.partial(
        pl.kernel,
        mesh=mesh,
        out_type=jax.ShapeDtypeStruct((B, Q, H), table.dtype),
        scratch_types=[pltpu.SemaphoreType.DMA],
    )
    def bcast(table_hbm, out_hbm, sem):
        wid = lax.axis_index("s") * NC + lax.axis_index("c")
        base = wid * b_per_w
        copies = [
            pltpu.async_copy(table_hbm, out_hbm.at[base + j], sem)
            for j in range(b_per_w)
        ]
        for c in copies:
            c.wait()

    return bcast(table)
